# initial kernel scaffold (unmeasured)
import jax
import jax.numpy as jnp
from jax import lax
from jax.experimental import pallas as pl
from jax.experimental.pallas import tpu as pltpu


def kernel(
    x,
):
    def body(*refs):
        pass

    out_shape = jax.ShapeDtypeStruct(..., jnp.float32)
    return pl.pallas_call(body, out_shape=out_shape)(...)


# baseline (device time: 6775 ns/iter reference)
import jax
import jax.numpy as jnp
from jax import lax
from jax.experimental import pallas as pl
from jax.experimental.pallas import tpu as pltpu

N_DEV = 32

TOP, BOT = 0, 1


def kernel(x):
    m, n = x.shape

    def body(x_ref, out_ref, halo_ref, send_sems, recv_sems):
        my = lax.axis_index("i")
        has_left = my > 0
        has_right = my < N_DEV - 1

        halo_ref[...] = jnp.zeros_like(halo_ref)

        barrier = pltpu.get_barrier_semaphore()

        @pl.when(has_left)
        def _():
            pl.semaphore_signal(
                barrier, inc=1,
                device_id=(my - 1,), device_id_type=pl.DeviceIdType.MESH,
            )

        @pl.when(has_right)
        def _():
            pl.semaphore_signal(
                barrier, inc=1,
                device_id=(my + 1,), device_id_type=pl.DeviceIdType.MESH,
            )

        @pl.when(has_left)
        def _():
            pl.semaphore_wait(barrier, 1)

        @pl.when(has_right)
        def _():
            pl.semaphore_wait(barrier, 1)

        send_left = pltpu.make_async_remote_copy(
            src_ref=x_ref.at[pl.ds(0, 1)],
            dst_ref=halo_ref.at[BOT],
            send_sem=send_sems.at[0],
            recv_sem=recv_sems.at[BOT],
            device_id=(my - 1,),
            device_id_type=pl.DeviceIdType.MESH,
        )
        send_right = pltpu.make_async_remote_copy(
            src_ref=x_ref.at[pl.ds(m - 1, 1)],
            dst_ref=halo_ref.at[TOP],
            send_sem=send_sems.at[1],
            recv_sem=recv_sems.at[TOP],
            device_id=(my + 1,),
            device_id_type=pl.DeviceIdType.MESH,
        )

        @pl.when(has_left)
        def _():
            send_left.start()

        @pl.when(has_right)
        def _():
            send_right.start()

        out_ref[pl.ds(1, m - 2), :] = (
            0.25 * x_ref[pl.ds(0, m - 2), :]
            + 0.5 * x_ref[pl.ds(1, m - 2), :]
            + 0.25 * x_ref[pl.ds(2, m - 2), :]
        )

        @pl.when(has_left)
        def _():
            send_left.wait_send()
            pltpu.make_async_remote_copy(
                src_ref=x_ref.at[pl.ds(m - 1, 1)],
                dst_ref=halo_ref.at[TOP],
                send_sem=send_sems.at[1],
                recv_sem=recv_sems.at[TOP],
                device_id=(my - 1,),
                device_id_type=pl.DeviceIdType.MESH,
            ).wait_recv()

        @pl.when(has_right)
        def _():
            send_right.wait_send()
            pltpu.make_async_remote_copy(
                src_ref=x_ref.at[pl.ds(0, 1)],
                dst_ref=halo_ref.at[BOT],
                send_sem=send_sems.at[0],
                recv_sem=recv_sems.at[BOT],
                device_id=(my + 1,),
                device_id_type=pl.DeviceIdType.MESH,
            ).wait_recv()

        top_row = jnp.where(
            my == 0,
            x_ref[pl.ds(0, 1), :],
            0.25 * halo_ref[TOP]
            + 0.5 * x_ref[pl.ds(0, 1), :]
            + 0.25 * x_ref[pl.ds(1, 1), :],
        )
        out_ref[pl.ds(0, 1), :] = top_row

        bot_row = jnp.where(
            my == N_DEV - 1,
            x_ref[pl.ds(m - 1, 1), :],
            0.25 * x_ref[pl.ds(m - 2, 1), :]
            + 0.5 * x_ref[pl.ds(m - 1, 1), :]
            + 0.25 * halo_ref[BOT],
        )
        out_ref[pl.ds(m - 1, 1), :] = bot_row

    return pl.pallas_call(
        body,
        out_shape=jax.ShapeDtypeStruct((m, n), x.dtype),
        in_specs=[pl.BlockSpec(memory_space=pltpu.VMEM)],
        out_specs=pl.BlockSpec(memory_space=pltpu.VMEM),
        scratch_shapes=[
            pltpu.VMEM((2, 1, n), x.dtype),
            pltpu.SemaphoreType.DMA((2,)),
            pltpu.SemaphoreType.DMA((2,)),
        ],
        compiler_params=pltpu.CompilerParams(collective_id=0),
    )(x)


# device time: 6749 ns/iter; 1.0039x vs baseline; 1.0039x over previous
import jax
import jax.numpy as jnp
from jax import lax
from jax.experimental import pallas as pl
from jax.experimental.pallas import tpu as pltpu

N_DEV = 32

TOP, BOT = 0, 1


def kernel(x):
    m, n = x.shape

    def body(x_ref, out_ref, halo_ref, send_sems, recv_sems):
        my = lax.axis_index("i")
        has_left = my > 0
        has_right = my < N_DEV - 1

        halo_ref[...] = jnp.zeros_like(halo_ref)

        barrier = pltpu.get_barrier_semaphore()

        @pl.when(has_left)
        def _():
            pl.semaphore_signal(
                barrier, inc=1,
                device_id=(my - 1,), device_id_type=pl.DeviceIdType.MESH,
            )

        @pl.when(has_right)
        def _():
            pl.semaphore_signal(
                barrier, inc=1,
                device_id=(my + 1,), device_id_type=pl.DeviceIdType.MESH,
            )

        @pl.when(has_left)
        def _():
            pl.semaphore_wait(barrier, 1)

        @pl.when(has_right)
        def _():
            pl.semaphore_wait(barrier, 1)

        send_left = pltpu.make_async_remote_copy(
            src_ref=x_ref.at[pl.ds(0, 1)],
            dst_ref=halo_ref.at[BOT],
            send_sem=send_sems.at[0],
            recv_sem=recv_sems.at[BOT],
            device_id=(my - 1,),
            device_id_type=pl.DeviceIdType.MESH,
        )
        send_right = pltpu.make_async_remote_copy(
            src_ref=x_ref.at[pl.ds(m - 1, 1)],
            dst_ref=halo_ref.at[TOP],
            send_sem=send_sems.at[1],
            recv_sem=recv_sems.at[TOP],
            device_id=(my + 1,),
            device_id_type=pl.DeviceIdType.MESH,
        )

        @pl.when(has_left)
        def _():
            send_left.start()

        @pl.when(has_right)
        def _():
            send_right.start()

        out_ref[pl.ds(1, m - 2), :] = (
            0.25 * x_ref[pl.ds(0, m - 2), :]
            + 0.5 * x_ref[pl.ds(1, m - 2), :]
            + 0.25 * x_ref[pl.ds(2, m - 2), :]
        )
        top_partial = 0.5 * x_ref[pl.ds(0, 1), :] + 0.25 * x_ref[pl.ds(1, 1), :]
        bot_partial = 0.25 * x_ref[pl.ds(m - 2, 1), :] + 0.5 * x_ref[pl.ds(m - 1, 1), :]

        @pl.when(has_left)
        def _():
            pltpu.make_async_remote_copy(
                src_ref=x_ref.at[pl.ds(m - 1, 1)],
                dst_ref=halo_ref.at[TOP],
                send_sem=send_sems.at[1],
                recv_sem=recv_sems.at[TOP],
                device_id=(my - 1,),
                device_id_type=pl.DeviceIdType.MESH,
            ).wait_recv()

        out_ref[pl.ds(0, 1), :] = jnp.where(
            my == 0,
            x_ref[pl.ds(0, 1), :],
            top_partial + 0.25 * halo_ref[TOP],
        )

        @pl.when(has_right)
        def _():
            pltpu.make_async_remote_copy(
                src_ref=x_ref.at[pl.ds(0, 1)],
                dst_ref=halo_ref.at[BOT],
                send_sem=send_sems.at[0],
                recv_sem=recv_sems.at[BOT],
                device_id=(my + 1,),
                device_id_type=pl.DeviceIdType.MESH,
            ).wait_recv()

        out_ref[pl.ds(m - 1, 1), :] = jnp.where(
            my == N_DEV - 1,
            x_ref[pl.ds(m - 1, 1), :],
            bot_partial + 0.25 * halo_ref[BOT],
        )

        @pl.when(has_left)
        def _():
            send_left.wait_send()

        @pl.when(has_right)
        def _():
            send_right.wait_send()

    return pl.pallas_call(
        body,
        out_shape=jax.ShapeDtypeStruct((m, n), x.dtype),
        in_specs=[pl.BlockSpec(memory_space=pltpu.VMEM)],
        out_specs=pl.BlockSpec(memory_space=pltpu.VMEM),
        scratch_shapes=[
            pltpu.VMEM((2, 1, n), x.dtype),
            pltpu.SemaphoreType.DMA((2,)),
            pltpu.SemaphoreType.DMA((2,)),
        ],
        compiler_params=pltpu.CompilerParams(collective_id=0),
    )(x)
